# baseline (device time: 155210 ns/iter reference)
import jax
import jax.numpy as jnp
from jax import lax
from jax.experimental import pallas as pl
from jax.experimental.pallas import tpu as pltpu

N_DEV = 4


def kernel(x, w_mat):
    m_per, k = x.shape
    _, n_per = w_mat.shape

    def body(x_ref, w_ref, out_ref, comm_ref, send_sems, recv_sems):
        my_pos = lax.axis_index("i")
        left = (my_pos - 1) % N_DEV
        right = (my_pos + 1) % N_DEV

        barrier_sem = pltpu.get_barrier_semaphore()
        for nbr in [left, right]:
            pl.semaphore_signal(
                barrier_sem, inc=1,
                device_id=(nbr,), device_id_type=pl.DeviceIdType.MESH,
            )
        pl.semaphore_wait(barrier_sem, 2)

        comm_ref[0, :, :] = x_ref[:, :]
        out_ref[pl.ds(my_pos * m_per, m_per), :] = jnp.dot(
            x_ref[:, :], w_ref[:, :], preferred_element_type=jnp.float32
        )

        for h in range(N_DEV - 1):
            send_slot = h % 2
            recv_slot = (h + 1) % 2
            rdma = pltpu.make_async_remote_copy(
                src_ref=comm_ref.at[send_slot],
                dst_ref=comm_ref.at[recv_slot],
                send_sem=send_sems.at[send_slot],
                recv_sem=recv_sems.at[recv_slot],
                device_id=(right,),
                device_id_type=pl.DeviceIdType.MESH,
            )
            rdma.start()
            rdma.wait()

            origin = (my_pos - h - 1) % N_DEV
            out_ref[pl.ds(origin * m_per, m_per), :] = jnp.dot(
                comm_ref[recv_slot, :, :], w_ref[:, :],
                preferred_element_type=jnp.float32,
            )

    return pl.pallas_call(
        body,
        out_shape=jax.ShapeDtypeStruct((N_DEV * m_per, n_per), jnp.float32),
        in_specs=[
            pl.BlockSpec(memory_space=pltpu.VMEM),
            pl.BlockSpec(memory_space=pltpu.VMEM),
        ],
        out_specs=pl.BlockSpec(memory_space=pltpu.VMEM),
        scratch_shapes=[
            pltpu.VMEM((2, m_per, k), jnp.float32),
            pltpu.SemaphoreType.DMA((2,)),
            pltpu.SemaphoreType.DMA((2,)),
        ],
        compiler_params=pltpu.CompilerParams(collective_id=0),
    )(x, w_mat)


# device time: 83763 ns/iter; 1.8530x vs baseline; 1.8530x over previous
import jax
import jax.numpy as jnp
from jax import lax
from jax.experimental import pallas as pl
from jax.experimental.pallas import tpu as pltpu

N_DEV = 4


def kernel(x, w_mat):
    m_per, k = x.shape
    _, n_per = w_mat.shape
    half = m_per // 2

    def body(x_ref, w_ref, out_ref, fwd_ref, bwd_ref,
             fsend_sems, frecv_sems, bsend_sems, brecv_sems):
        my_pos = lax.axis_index("i")
        left = (my_pos - 1) % N_DEV
        right = (my_pos + 1) % N_DEV

        barrier_sem = pltpu.get_barrier_semaphore()
        for nbr in [left, right]:
            pl.semaphore_signal(
                barrier_sem, inc=1,
                device_id=(nbr,), device_id_type=pl.DeviceIdType.MESH,
            )
        pl.semaphore_wait(barrier_sem, 2)

        fwd_ref[0, :, :] = x_ref[:half, :]
        bwd_ref[0, :, :] = x_ref[half:, :]

        def half_block(row, src_ref):
            out_ref[pl.ds(row, half), :] = jnp.dot(
                src_ref[:, :], w_ref[:, :],
                preferred_element_type=jnp.float32,
            )

        def hop_compute(h, s):
            half_block(((my_pos - h) % N_DEV) * m_per, fwd_ref.at[s])
            half_block(((my_pos + h) % N_DEV) * m_per + half, bwd_ref.at[s])

        for h in range(N_DEV - 1):
            s = h % 2
            r = (h + 1) % 2
            fwd = pltpu.make_async_remote_copy(
                src_ref=fwd_ref.at[s], dst_ref=fwd_ref.at[r],
                send_sem=fsend_sems.at[s], recv_sem=frecv_sems.at[r],
                device_id=(right,), device_id_type=pl.DeviceIdType.MESH,
            )
            bwd = pltpu.make_async_remote_copy(
                src_ref=bwd_ref.at[s], dst_ref=bwd_ref.at[r],
                send_sem=bsend_sems.at[s], recv_sem=brecv_sems.at[r],
                device_id=(left,), device_id_type=pl.DeviceIdType.MESH,
            )
            fwd.start()
            bwd.start()

            if h == 0:
                half_block(my_pos * m_per, x_ref.at[:half])
                half_block(my_pos * m_per + half, x_ref.at[half:])
            else:
                hop_compute(h, s)

            fwd.wait()
            bwd.wait()

        hop_compute(3, 1)

    return pl.pallas_call(
        body,
        out_shape=jax.ShapeDtypeStruct((N_DEV * m_per, n_per), jnp.float32),
        in_specs=[
            pl.BlockSpec(memory_space=pltpu.VMEM),
            pl.BlockSpec(memory_space=pltpu.VMEM),
        ],
        out_specs=pl.BlockSpec(memory_space=pltpu.VMEM),
        scratch_shapes=[
            pltpu.VMEM((2, half, k), jnp.float32),
            pltpu.VMEM((2, half, k), jnp.float32),
            pltpu.SemaphoreType.DMA((2,)),
            pltpu.SemaphoreType.DMA((2,)),
            pltpu.SemaphoreType.DMA((2,)),
            pltpu.SemaphoreType.DMA((2,)),
        ],
        compiler_params=pltpu.CompilerParams(collective_id=0),
    )(x, w_mat)


# device time: 79946 ns/iter; 1.9414x vs baseline; 1.0477x over previous
import functools

import jax
import jax.numpy as jnp
from jax import lax
from jax.experimental import pallas as pl
from jax.experimental.pallas import tpu as pltpu

N_DEV = 4
H = N_DEV - 1
P = 2


def kernel(x, w_mat):
    m_per, k = x.shape
    _, n_per = w_mat.shape
    half = m_per // 2
    piece = half // P

    def body(x_ref, w_ref, out_ref, fwd_ref, bwd_ref,
             fsend, frecv, bsend, brecv):
        my_pos = lax.axis_index("i")
        left = (my_pos - 1) % N_DEV
        right = (my_pos + 1) % N_DEV

        barrier_sem = pltpu.get_barrier_semaphore()
        for nbr in [left, right]:
            pl.semaphore_signal(
                barrier_sem, inc=1,
                device_id=(nbr,), device_id_type=pl.DeviceIdType.MESH,
            )
        pl.semaphore_wait(barrier_sem, 2)

        pending = []

        def rdma(src, dst, ssem, rsem, target):
            d = pltpu.make_async_remote_copy(
                src_ref=src, dst_ref=dst, send_sem=ssem, recv_sem=rsem,
                device_id=(target,), device_id_type=pl.DeviceIdType.MESH,
            )
            d.start()
            pending.append(d)

        for p in range(P):
            rdma(x_ref.at[pl.ds(p * piece, piece)], fwd_ref.at[0, p],
                 fsend.at[0, p], frecv.at[0, p], right)
            rdma(x_ref.at[pl.ds(half + p * piece, piece)], bwd_ref.at[0, p],
                 bsend.at[0, p], brecv.at[0, p], left)

        out_ref[pl.ds(my_pos * m_per, m_per), :] = jnp.dot(
            x_ref[:, :], w_ref[:, :], preferred_element_type=jnp.float32
        )

        def wait_recv(buf, ssem, rsem, h, p, target):
            d = pltpu.make_async_remote_copy(
                src_ref=buf.at[h, p], dst_ref=buf.at[h, p],
                send_sem=ssem.at[h, p], recv_sem=rsem.at[h, p],
                device_id=(target,), device_id_type=pl.DeviceIdType.MESH,
            )
            d.wait_recv()

        for h in range(H):
            for p in range(P):
                wait_recv(fwd_ref, fsend, frecv, h, p, right)
                if h + 1 < H:
                    rdma(fwd_ref.at[h, p], fwd_ref.at[h + 1, p],
                         fsend.at[h + 1, p], frecv.at[h + 1, p], right)
                wait_recv(bwd_ref, bsend, brecv, h, p, left)
                if h + 1 < H:
                    rdma(bwd_ref.at[h, p], bwd_ref.at[h + 1, p],
                         bsend.at[h + 1, p], brecv.at[h + 1, p], left)

                fo = (my_pos - 1 - h) % N_DEV
                out_ref[pl.ds(fo * m_per + p * piece, piece), :] = jnp.dot(
                    fwd_ref[h, p, :, :], w_ref[:, :],
                    preferred_element_type=jnp.float32,
                )
                bo = (my_pos + 1 + h) % N_DEV
                out_ref[pl.ds(bo * m_per + half + p * piece, piece), :] = (
                    jnp.dot(
                        bwd_ref[h, p, :, :], w_ref[:, :],
                        preferred_element_type=jnp.float32,
                    )
                )

        for d in pending:
            d.wait_send()

        @functools.partial(
            pl.run_scoped, second_barrier=pltpu.SemaphoreType.REGULAR
        )
        def _(second_barrier):
            for nbr in [left, right]:
                pl.semaphore_signal(
                    second_barrier, inc=1,
                    device_id=(nbr,), device_id_type=pl.DeviceIdType.MESH,
                )
            pl.semaphore_wait(second_barrier, 2)

    return pl.pallas_call(
        body,
        out_shape=jax.ShapeDtypeStruct((N_DEV * m_per, n_per), jnp.float32),
        in_specs=[
            pl.BlockSpec(memory_space=pltpu.VMEM),
            pl.BlockSpec(memory_space=pltpu.VMEM),
        ],
        out_specs=pl.BlockSpec(memory_space=pltpu.VMEM),
        scratch_shapes=[
            pltpu.VMEM((H, P, piece, k), jnp.float32),
            pltpu.VMEM((H, P, piece, k), jnp.float32),
            pltpu.SemaphoreType.DMA((H, P)),
            pltpu.SemaphoreType.DMA((H, P)),
            pltpu.SemaphoreType.DMA((H, P)),
            pltpu.SemaphoreType.DMA((H, P)),
        ],
        compiler_params=pltpu.CompilerParams(collective_id=0),
    )(x, w_mat)
